# baseline (speedup 1.0000x reference)
import jax
import jax.numpy as jnp
from jax.experimental import pallas as pl
from jax.experimental.pallas import tpu as pltpu

_T = 0.07


def _body(lbl_ref, q_ref, k_ref, slab_ref, out_ref):
    i = pl.program_id(0)
    n = pl.num_programs(0)
    qrow = q_ref[0]
    krow = k_ref[0]
    lpos = jnp.sum(qrow * krow)
    s = jax.lax.dot_general(
        qrow, slab_ref[0], (((1,), (0,)), ((), ())),
        preferred_element_type=jnp.float32,
    )
    m = jnp.maximum(jnp.max(s), lpos)
    se = jnp.sum(jnp.exp(s - m)) + jnp.exp(lpos - m)
    lse = m + jnp.log(se)
    contrib = lse - lpos

    @pl.when(i == 0)
    def _init():
        out_ref[...] = jnp.zeros_like(out_ref)

    out_ref[...] = out_ref[...] + contrib / n


def kernel(q, k, weight, cls_labels, queue):
    del weight
    n, d = q.shape
    c, _, kq = queue.shape

    labels = cls_labels.astype(jnp.int32)
    iota_n = jnp.arange(n, dtype=jnp.int32)
    eq = labels[:, None] == labels[None, :]
    rank = jnp.sum(eq & (iota_n[None, :] < iota_n[:, None]), axis=1)
    hist = jnp.sum(
        labels[:, None] == jnp.arange(c, dtype=jnp.int32)[None, :], axis=0
    ).astype(jnp.int32)
    offs = jnp.concatenate(
        [jnp.zeros((1,), jnp.int32), jnp.cumsum(hist)[:-1]]
    )
    pos = offs[labels] + rank.astype(jnp.int32)
    order = jnp.zeros((n,), jnp.int32).at[pos].set(iota_n)

    slabels = labels[order]
    qs = (q[order] * (1.0 / _T)).reshape(n, 1, d)
    ks = k[order].reshape(n, 1, d)

    grid_spec = pltpu.PrefetchScalarGridSpec(
        num_scalar_prefetch=1,
        grid=(n,),
        in_specs=[
            pl.BlockSpec((1, 1, d), lambda i, lbl: (i, 0, 0)),
            pl.BlockSpec((1, 1, d), lambda i, lbl: (i, 0, 0)),
            pl.BlockSpec((1, d, kq), lambda i, lbl: (lbl[i], 0, 0)),
        ],
        out_specs=pl.BlockSpec((1, 1), lambda i, lbl: (0, 0)),
    )
    out = pl.pallas_call(
        _body,
        grid_spec=grid_spec,
        out_shape=jax.ShapeDtypeStruct((1, 1), jnp.float32),
    )(slabels, qs, ks, queue)
    return out[0, 0]


# speedup vs baseline: 1.2230x; 1.2230x over previous
import jax
import jax.numpy as jnp
from jax.experimental import pallas as pl
from jax.experimental.pallas import tpu as pltpu

_T = 0.07


def _body(lbl_ref, q_ref, k_ref, slab_ref, out_ref, lneg_ref, lp_ref):
    i = pl.program_id(0)
    n = pl.num_programs(0)
    qrow = q_ref[0] * (1.0 / _T)
    s = jax.lax.dot_general(
        qrow, slab_ref[0], (((1,), (0,)), ((), ())),
        preferred_element_type=jnp.float32,
    )
    lneg_ref[pl.ds(i, 1), :] = s
    lp_ref[pl.ds(i, 1), :] = qrow * k_ref[0]

    @pl.when(i == n - 1)
    def _finish():
        lp = jnp.sum(lp_ref[...], axis=1, keepdims=True)
        ln = lneg_ref[...]
        m = jnp.maximum(jnp.max(ln, axis=1, keepdims=True), lp)
        se = jnp.sum(jnp.exp(ln - m), axis=1, keepdims=True) + jnp.exp(lp - m)
        lse = m + jnp.log(se)
        loss = jnp.mean(lse - lp)
        out_ref[...] = jnp.full((1, 1), loss, dtype=jnp.float32)


def kernel(q, k, weight, cls_labels, queue):
    del weight
    n, d = q.shape
    c, _, kq = queue.shape

    labels = cls_labels.astype(jnp.int32)
    iota_n = jnp.arange(n, dtype=jnp.int32)
    lab_r, lab_c = labels[:, None], labels[None, :]
    lt = iota_n[None, :] < iota_n[:, None]
    pos = jnp.sum((lab_c < lab_r) | ((lab_c == lab_r) & lt), axis=1)
    order = jnp.sum(
        (pos[:, None] == iota_n[None, :]) * iota_n[:, None],
        axis=0,
    ).astype(jnp.int32)

    slabels = labels[order]
    q3 = q[order].reshape(n, 1, d)
    k3 = k[order].reshape(n, 1, d)

    grid_spec = pltpu.PrefetchScalarGridSpec(
        num_scalar_prefetch=1,
        grid=(n,),
        in_specs=[
            pl.BlockSpec((1, 1, d), lambda i, lbl: (i, 0, 0)),
            pl.BlockSpec((1, 1, d), lambda i, lbl: (i, 0, 0)),
            pl.BlockSpec((1, d, kq), lambda i, lbl: (lbl[i], 0, 0)),
        ],
        out_specs=pl.BlockSpec((1, 1), lambda i, lbl: (0, 0)),
        scratch_shapes=[
            pltpu.VMEM((n, kq), jnp.float32),
            pltpu.VMEM((n, d), jnp.float32),
        ],
    )
    out = pl.pallas_call(
        _body,
        grid_spec=grid_spec,
        out_shape=jax.ShapeDtypeStruct((1, 1), jnp.float32),
    )(slabels, q3, k3, queue)
    return out[0, 0]


# speedup vs baseline: 2.2922x; 1.8743x over previous
import jax
import jax.numpy as jnp
from jax.experimental import pallas as pl
from jax.experimental.pallas import tpu as pltpu

_T = 0.07
_NBUF = 4


def _body(segcls_ref, segid_ref, first_ref, nseg_ref,
          q_ref, k_ref, queue_ref, out_ref, bufs_ref, lneg_ref, sem_ref):
    n, d = q_ref.shape
    nseg = nseg_ref[0]

    def _issue(s):
        c = segcls_ref[s]
        pltpu.make_async_copy(
            queue_ref.at[c], bufs_ref.at[s % _NBUF], sem_ref.at[s % _NBUF]
        ).start()

    _issue(0)

    @pl.when(nseg > 1)
    def _i1():
        _issue(1)

    @pl.when(nseg > 2)
    def _i2():
        _issue(2)

    def _step(i, carry):
        s = segid_ref[i]

        @pl.when(first_ref[i] == 1)
        def _seg_start():
            pltpu.make_async_copy(
                queue_ref.at[segcls_ref[s]],
                bufs_ref.at[s % _NBUF],
                sem_ref.at[s % _NBUF],
            ).wait()

            @pl.when(s + 3 < nseg)
            def _prefetch():
                _issue(s + 3)

        qrow = q_ref[pl.ds(i, 1), :] * (1.0 / _T)
        res = jax.lax.dot_general(
            qrow, bufs_ref[s % _NBUF], (((1,), (0,)), ((), ())),
            preferred_element_type=jnp.float32,
        )
        lneg_ref[pl.ds(i, 1), :] = res
        return carry

    jax.lax.fori_loop(0, n, _step, 0)

    lp = jnp.sum(q_ref[...] * k_ref[...], axis=1, keepdims=True) * (1.0 / _T)
    ln = lneg_ref[...]
    m = jnp.maximum(jnp.max(ln, axis=1, keepdims=True), lp)
    se = jnp.sum(jnp.exp(ln - m), axis=1, keepdims=True) + jnp.exp(lp - m)
    lse = m + jnp.log(se)
    loss = jnp.mean(lse - lp)
    out_ref[...] = jnp.full((1, 1), loss, dtype=jnp.float32)


def kernel(q, k, weight, cls_labels, queue):
    del weight
    n, d = q.shape
    c, _, kq = queue.shape

    labels = cls_labels.astype(jnp.int32)
    iota_n = jnp.arange(n, dtype=jnp.int32)
    lab_r, lab_c = labels[:, None], labels[None, :]
    lt = iota_n[None, :] < iota_n[:, None]
    pos = jnp.sum((lab_c < lab_r) | ((lab_c == lab_r) & lt), axis=1)
    order = jnp.sum(
        (pos[:, None] == iota_n[None, :]) * iota_n[:, None], axis=0
    ).astype(jnp.int32)

    slabels = labels[order]
    qs = q[order]
    ks = k[order]

    newseg = jnp.concatenate(
        [jnp.ones((1,), jnp.int32),
         (slabels[1:] != slabels[:-1]).astype(jnp.int32)]
    )
    segid = jnp.cumsum(newseg) - 1
    nsegarr = segid[-1:] + 1
    segcls = jnp.zeros((n,), jnp.int32).at[segid].set(slabels)

    grid_spec = pltpu.PrefetchScalarGridSpec(
        num_scalar_prefetch=4,
        grid=(1,),
        in_specs=[
            pl.BlockSpec((n, d), lambda i, *_: (0, 0)),
            pl.BlockSpec((n, d), lambda i, *_: (0, 0)),
            pl.BlockSpec(memory_space=pl.ANY),
        ],
        out_specs=pl.BlockSpec((1, 1), lambda i, *_: (0, 0)),
        scratch_shapes=[
            pltpu.VMEM((_NBUF, d, kq), jnp.float32),
            pltpu.VMEM((n, kq), jnp.float32),
            pltpu.SemaphoreType.DMA((_NBUF,)),
        ],
    )
    out = pl.pallas_call(
        _body,
        grid_spec=grid_spec,
        out_shape=jax.ShapeDtypeStruct((1, 1), jnp.float32),
    )(segcls, segid, newseg, nsegarr, qs, ks, queue)
    return out[0, 0]


# speedup vs baseline: 2.4635x; 1.0747x over previous
import jax
import jax.numpy as jnp
from jax.experimental import pallas as pl
from jax.experimental.pallas import tpu as pltpu

_T = 0.07
_NBUF = 8


def _body(segcls_ref, segid_ref, first_ref, nseg_ref, order_ref,
          q_ref, k_ref, queue_ref, out_ref, bufs_ref, lneg_ref, sem_ref):
    n, d = q_ref.shape
    nseg = nseg_ref[0]

    def _issue(s):
        c = segcls_ref[s]
        pltpu.make_async_copy(
            queue_ref.at[c], bufs_ref.at[s % _NBUF], sem_ref.at[s % _NBUF]
        ).start()

    _issue(0)
    for j in range(1, _NBUF - 1):
        @pl.when(nseg > j)
        def _ij(j=j):
            _issue(j)

    def _step(i, carry):
        s = segid_ref[i]
        r = order_ref[i]

        @pl.when(first_ref[i] == 1)
        def _seg_start():
            pltpu.make_async_copy(
                queue_ref.at[segcls_ref[s]],
                bufs_ref.at[s % _NBUF],
                sem_ref.at[s % _NBUF],
            ).wait()

            @pl.when(s + _NBUF - 1 < nseg)
            def _prefetch():
                _issue(s + _NBUF - 1)

        qrow = q_ref[pl.ds(r, 1), :] * (1.0 / _T)
        res = jax.lax.dot_general(
            qrow, bufs_ref[s % _NBUF], (((1,), (0,)), ((), ())),
            preferred_element_type=jnp.float32,
        )
        lneg_ref[pl.ds(r, 1), :] = res
        return carry

    jax.lax.fori_loop(0, n, _step, 0)

    lp = jnp.sum(q_ref[...] * k_ref[...], axis=1, keepdims=True) * (1.0 / _T)
    ln = lneg_ref[...]
    m = jnp.maximum(jnp.max(ln, axis=1, keepdims=True), lp)
    se = jnp.sum(jnp.exp(ln - m), axis=1, keepdims=True) + jnp.exp(lp - m)
    lse = m + jnp.log(se)
    loss = jnp.mean(lse - lp)
    out_ref[...] = jnp.full((1, 1), loss, dtype=jnp.float32)


def kernel(q, k, weight, cls_labels, queue):
    del weight
    n, d = q.shape
    c, _, kq = queue.shape

    labels = cls_labels.astype(jnp.int32)
    iota_n = jnp.arange(n, dtype=jnp.int32)
    lab_r, lab_c = labels[:, None], labels[None, :]
    lt = iota_n[None, :] < iota_n[:, None]
    pos = jnp.sum((lab_c < lab_r) | ((lab_c == lab_r) & lt), axis=1)
    order = jnp.sum(
        (pos[:, None] == iota_n[None, :]) * iota_n[:, None], axis=0
    ).astype(jnp.int32)

    slabels = labels[order]
    newseg = jnp.concatenate(
        [jnp.ones((1,), jnp.int32),
         (slabels[1:] != slabels[:-1]).astype(jnp.int32)]
    )
    segid = jnp.cumsum(newseg) - 1
    nsegarr = segid[-1:] + 1
    segcls = jnp.zeros((n,), jnp.int32).at[segid].set(slabels)

    grid_spec = pltpu.PrefetchScalarGridSpec(
        num_scalar_prefetch=5,
        grid=(1,),
        in_specs=[
            pl.BlockSpec((n, d), lambda i, *_: (0, 0)),
            pl.BlockSpec((n, d), lambda i, *_: (0, 0)),
            pl.BlockSpec(memory_space=pl.ANY),
        ],
        out_specs=pl.BlockSpec((1, 1), lambda i, *_: (0, 0)),
        scratch_shapes=[
            pltpu.VMEM((_NBUF, d, kq), jnp.float32),
            pltpu.VMEM((n, kq), jnp.float32),
            pltpu.SemaphoreType.DMA((_NBUF,)),
        ],
    )
    out = pl.pallas_call(
        _body,
        grid_spec=grid_spec,
        out_shape=jax.ShapeDtypeStruct((1, 1), jnp.float32),
    )(segcls, segid, newseg, nsegarr, order, q, k, queue)
    return out[0, 0]


# speedup vs baseline: 2.7013x; 1.0965x over previous
import jax
import jax.numpy as jnp
from jax.experimental import pallas as pl
from jax.experimental.pallas import tpu as pltpu

_T = 0.07
_NBUF = 8


def _body(labr_ref, labc_ref, q_ref, k_ref, queue_ref, out_ref,
          bufs_ref, lneg_ref, pvm_ref, psm_ref, sem_ref, psem_ref):
    n, d = q_ref.shape
    i32 = jnp.int32

    labr = labr_ref[...]
    labc = labc_ref[...]
    ior = jax.lax.broadcasted_iota(i32, (n, n), 0)
    ioc = jax.lax.broadcasted_iota(i32, (n, n), 1)
    eqm = (labc == labr)
    ltm = (labc > labr) | (eqm & (ior > ioc))
    pos_c = jnp.sum(ltm.astype(i32), axis=1, keepdims=True)
    pos_r = jnp.sum(ltm.astype(i32), axis=0, keepdims=True)
    pos_r = (n - 1) - pos_r
    before = eqm & (ior > ioc)
    rf_c = (jnp.sum(before.astype(i32), axis=1, keepdims=True) == 0)
    after = eqm & (ior < ioc)
    rf_r = (jnp.sum(after.astype(i32), axis=0, keepdims=True) == 0)
    em = (pos_c == ioc)
    order_r = jnp.sum(jnp.where(em, ior, 0), axis=0, keepdims=True)
    first_r = jnp.sum((em & rf_c).astype(i32), axis=0, keepdims=True)
    segid_r = jnp.sum((rf_c & (pos_c <= ioc)).astype(i32),
                      axis=0, keepdims=True) - 1
    nseg = jnp.sum(rf_c.astype(i32))
    nseg_r = jnp.zeros((1, n), i32) + nseg
    segrow_c = jnp.sum((rf_r & (pos_r <= pos_c)).astype(i32),
                       axis=1, keepdims=True) - 1
    segcls_r = jnp.sum(jnp.where(rf_c & (segrow_c == ioc), labc, 0),
                       axis=0, keepdims=True)

    pvm_ref[pl.ds(0, 1), :] = segcls_r
    pvm_ref[pl.ds(1, 1), :] = segid_r
    pvm_ref[pl.ds(2, 1), :] = first_r
    pvm_ref[pl.ds(3, 1), :] = nseg_r
    pvm_ref[pl.ds(4, 1), :] = order_r
    copy = pltpu.make_async_copy(pvm_ref, psm_ref, psem_ref)
    copy.start()
    copy.wait()

    nsegv = psm_ref[3, 0]

    def _issue(s):
        c = psm_ref[0, s]
        pltpu.make_async_copy(
            queue_ref.at[c], bufs_ref.at[s % _NBUF], sem_ref.at[s % _NBUF]
        ).start()

    _issue(0)
    for j in range(1, _NBUF - 1):
        @pl.when(nsegv > j)
        def _ij(j=j):
            _issue(j)

    def _step(i, carry):
        s = psm_ref[1, i]
        r = psm_ref[4, i]

        @pl.when(psm_ref[2, i] == 1)
        def _seg_start():
            pltpu.make_async_copy(
                queue_ref.at[psm_ref[0, s]],
                bufs_ref.at[s % _NBUF],
                sem_ref.at[s % _NBUF],
            ).wait()

            @pl.when(s + _NBUF - 1 < nsegv)
            def _prefetch():
                _issue(s + _NBUF - 1)

        qrow = q_ref[pl.ds(r, 1), :] * (1.0 / _T)
        res = jax.lax.dot_general(
            qrow, bufs_ref[s % _NBUF], (((1,), (0,)), ((), ())),
            preferred_element_type=jnp.float32,
        )
        lneg_ref[pl.ds(r, 1), :] = res
        return carry

    jax.lax.fori_loop(0, n, _step, 0)

    lp = jnp.sum(q_ref[...] * k_ref[...], axis=1, keepdims=True) * (1.0 / _T)
    ln = lneg_ref[...]
    m = jnp.maximum(jnp.max(ln, axis=1, keepdims=True), lp)
    se = jnp.sum(jnp.exp(ln - m), axis=1, keepdims=True) + jnp.exp(lp - m)
    lse = m + jnp.log(se)
    loss = jnp.mean(lse - lp)
    out_ref[...] = jnp.full((1, 1), loss, dtype=jnp.float32)


def kernel(q, k, weight, cls_labels, queue):
    del weight
    n, d = q.shape
    c, _, kq = queue.shape

    labels = cls_labels.astype(jnp.int32)
    labr = labels.reshape(1, n)
    labc = labels.reshape(n, 1)

    out = pl.pallas_call(
        _body,
        grid=(1,),
        in_specs=[
            pl.BlockSpec((1, n), lambda i: (0, 0)),
            pl.BlockSpec((n, 1), lambda i: (0, 0)),
            pl.BlockSpec((n, d), lambda i: (0, 0)),
            pl.BlockSpec((n, d), lambda i: (0, 0)),
            pl.BlockSpec(memory_space=pl.ANY),
        ],
        out_specs=pl.BlockSpec((1, 1), lambda i: (0, 0)),
        scratch_shapes=[
            pltpu.VMEM((_NBUF, d, kq), jnp.float32),
            pltpu.VMEM((n, kq), jnp.float32),
            pltpu.VMEM((8, n), jnp.int32),
            pltpu.SMEM((8, n), jnp.int32),
            pltpu.SemaphoreType.DMA((_NBUF,)),
            pltpu.SemaphoreType.DMA,
        ],
        out_shape=jax.ShapeDtypeStruct((1, 1), jnp.float32),
    )(labr, labc, q, k, queue)
    return out[0, 0]
